# baseline (speedup 1.0000x reference)
import functools

import jax
import jax.numpy as jnp
from jax import lax
from jax.experimental import pallas as pl
from jax.experimental.pallas import tpu as pltpu
from jax.experimental.pallas import tpu_sc as plsc

B, N, F = 4096, 64, 9
VOCAB, EMB = 100, 128
R = B * N
NC, NS = 2, 16
NW = NC * NS
RW = R // NW
C = 128
NCHUNK = RW // C
NBLK = NW * NCHUNK
NP = 4
NT = NP + 1


def _pair_body(w0_ref, w1_ref, out_ref):
    out_ref[0, 0] = w1_ref[0] + w0_ref[0, 0]


@functools.cache
def _build_pair_tables():
    return pl.pallas_call(
        _pair_body,
        grid=(NP, VOCAB),
        in_specs=[
            pl.BlockSpec((1, 1, 1, EMB), lambda p, i: (p, i, 0, 0)),
            pl.BlockSpec((1, VOCAB, EMB), lambda p, i: (p, 0, 0)),
        ],
        out_specs=pl.BlockSpec((1, 1, VOCAB, EMB), lambda p, i: (p, i, 0, 0)),
        out_shape=jax.ShapeDtypeStruct((NP, VOCAB, VOCAB, EMB), jnp.float32),
    )


@functools.cache
def _build_sc_kernel():
    mesh = plsc.VectorSubcoreMesh(core_axis_name="c", subcore_axis_name="s")

    @functools.partial(
        pl.kernel,
        out_type=jax.ShapeDtypeStruct((R, EMB), jnp.float32),
        mesh=mesh,
        scratch_types=[
            pltpu.VMEM((2, NT, C), jnp.int32),
            pltpu.VMEM((NT, C, EMB), jnp.float32),
            pltpu.VMEM((2, C), jnp.int32),
            pltpu.VMEM_SHARED((2 * NS * C, EMB), jnp.float32),
            pltpu.SemaphoreType.DMA,
            pltpu.SemaphoreType.DMA,
            pltpu.SemaphoreType.DMA,
            pltpu.SemaphoreType.DMA,
            pltpu.SemaphoreType.DMA,
            pltpu.SemaphoreType.DMA,
            pltpu.SemaphoreType.DMA,
            pltpu.SemaphoreType.DMA,
            pltpu.SemaphoreType.DMA,
            pltpu.SemaphoreType.DMA,
            pltpu.SemaphoreType.DMA,
            pltpu.SemaphoreType.DMA,
            pltpu.SemaphoreType.DMA,
        ],
    )
    def emb_kernel(idx_hbm, pair_hbm, w8_hbm, zero_hbm, out_hbm,
                   idxv, gbuf, idconst, acc,
                   isem, osem, zsem, g0, g1, g2, g3, g4, s0, s1, s2, s3, s4):
        gsems = (g0, g1, g2, g3, g4)
        ssems = (s0, s1, s2, s3, s4)
        cid = lax.axis_index("c")
        sid = lax.axis_index("s")
        wid = sid * NC + cid
        lane = lax.broadcasted_iota(jnp.int32, (16,), 0)
        for b in range(2):
            for i in range(C // 16):
                idconst[b, pl.ds(i * 16, 16)] = lane + (b * NS * C + sid * C + i * 16)

        def table_ref(bank, f):
            src = pair_hbm if f < NP else w8_hbm
            return src.at[idxv.at[bank, f]]

        def fire_idx(ci):
            blk = jnp.minimum(wid * NCHUNK + ci, NBLK - 1)
            pltpu.async_copy(idx_hbm.at[blk], idxv.at[ci % 2], isem)

        def acc_slice(bank):
            return acc.at[pl.ds(bank * NS * C + sid * C, C), :]

        def out_slices(ci):
            blk = wid * NCHUNK + ci
            return acc_slice(ci % 2), out_hbm.at[pl.ds(blk * C, C), :]

        fire_idx(0)

        def chunk_body(ci, carry):
            bank = ci % 2
            pltpu.make_async_copy(idx_hbm.at[0], idxv.at[bank], isem).wait()
            @pl.when(ci >= 2)
            def _():
                src, dst = out_slices(ci - 2)
                pltpu.make_async_copy(src, dst, osem).wait()

            pltpu.async_copy(zero_hbm, acc_slice(bank), zsem)
            for f in range(NT):
                pltpu.async_copy(table_ref(bank, f), gbuf.at[f], gsems[f])
            fire_idx(ci + 1)
            pltpu.make_async_copy(zero_hbm, acc_slice(bank), zsem).wait()
            for f in range(NT):
                pltpu.make_async_copy(table_ref(bank, f), gbuf.at[f], gsems[f]).wait()
                pltpu.async_copy(gbuf.at[f], acc.at[idconst.at[bank]], ssems[f],
                                 add=True)
            for f in range(NT):
                pltpu.make_async_copy(gbuf.at[f], acc.at[idconst.at[bank]],
                                      ssems[f]).wait()
            src, dst = out_slices(ci)
            pltpu.async_copy(src, dst, osem)
            return carry

        lax.fori_loop(0, NCHUNK, chunk_body, 0)

        for ci in (NCHUNK - 2, NCHUNK - 1):
            src, dst = out_slices(ci)
            pltpu.make_async_copy(src, dst, osem).wait()
        pltpu.make_async_copy(idx_hbm.at[0], idxv.at[NCHUNK % 2], isem).wait()

    return emb_kernel


@jax.jit
def kernel(node_feature, W):
    idx = node_feature.astype(jnp.int32)
    pidx = (idx[..., 0:2 * NP:2] * VOCAB + idx[..., 1:2 * NP:2]
            + jnp.arange(NP, dtype=jnp.int32) * (VOCAB * VOCAB))
    allidx = jnp.concatenate([pidx, idx[..., 2 * NP:]], axis=-1)
    idx_all = (
        allidx.reshape(R, NT).T.reshape(NT, NBLK, C).transpose(1, 0, 2)
    )
    pairs = _build_pair_tables()(
        W[0:2 * NP:2].reshape(NP, VOCAB, 1, EMB), W[1:2 * NP:2]
    ).reshape(NP * VOCAB * VOCAB, EMB)
    zero = jnp.zeros((C, EMB), jnp.float32)
    out = _build_sc_kernel()(idx_all, pairs, W[2 * NP], zero)
    return out.reshape(B, N, EMB)


# speedup vs baseline: 1.2822x; 1.2822x over previous
import functools

import jax
import jax.numpy as jnp
from jax import lax
from jax.experimental import pallas as pl
from jax.experimental.pallas import tpu as pltpu
from jax.experimental.pallas import tpu_sc as plsc

B, N, F = 4096, 64, 9
VOCAB, EMB = 100, 128
R = B * N
NC, NS = 2, 16
NW = NC * NS
RW = R // NW
C = 128
NCHUNK = RW // C
NBLK = NW * NCHUNK
NP = 4
NT = NP + 1


def _pair_body(w0_ref, w1_ref, out_ref):
    out_ref[...] = w0_ref[...] + w1_ref[...]


@functools.cache
def _build_pair_tables():
    return pl.pallas_call(
        _pair_body,
        grid=(NP,),
        in_specs=[
            pl.BlockSpec((1, VOCAB, 1, EMB), lambda p: (p, 0, 0, 0)),
            pl.BlockSpec((1, 1, VOCAB, EMB), lambda p: (p, 0, 0, 0)),
        ],
        out_specs=pl.BlockSpec((1, VOCAB, VOCAB, EMB), lambda p: (p, 0, 0, 0)),
        out_shape=jax.ShapeDtypeStruct((NP, VOCAB, VOCAB, EMB), jnp.float32),
    )


@functools.cache
def _build_sc_kernel():
    mesh = plsc.VectorSubcoreMesh(core_axis_name="c", subcore_axis_name="s")

    @functools.partial(
        pl.kernel,
        out_type=jax.ShapeDtypeStruct((R, EMB), jnp.float32),
        mesh=mesh,
        scratch_types=[
            pltpu.VMEM((2, F, C), jnp.int32),
            pltpu.VMEM((NT, C), jnp.int32),
            pltpu.VMEM((NT, C, EMB), jnp.float32),
            pltpu.VMEM((2, C), jnp.int32),
            pltpu.VMEM_SHARED((2 * NS * C, EMB), jnp.float32),
            pltpu.SemaphoreType.DMA,
            pltpu.SemaphoreType.DMA,
            pltpu.SemaphoreType.DMA,
            pltpu.SemaphoreType.DMA,
            pltpu.SemaphoreType.DMA,
            pltpu.SemaphoreType.DMA,
            pltpu.SemaphoreType.DMA,
            pltpu.SemaphoreType.DMA,
            pltpu.SemaphoreType.DMA,
            pltpu.SemaphoreType.DMA,
            pltpu.SemaphoreType.DMA,
            pltpu.SemaphoreType.DMA,
        ],
    )
    def emb_kernel(nf_hbm, pair_hbm, w8_hbm, out_hbm,
                   nfv, idxv, gbuf, idconst, acc,
                   isem, osem, g0, g1, g2, g3, g4, s0, s1, s2, s3, s4):
        gsems = (g0, g1, g2, g3, g4)
        ssems = (s0, s1, s2, s3, s4)
        cid = lax.axis_index("c")
        sid = lax.axis_index("s")
        wid = sid * NC + cid
        lane = lax.broadcasted_iota(jnp.int32, (16,), 0)
        for b in range(2):
            for i in range(C // 16):
                idconst[b, pl.ds(i * 16, 16)] = lane + (b * NS * C + sid * C + i * 16)

        def table_ref(f):
            src = pair_hbm if f < NP else w8_hbm
            return src.at[idxv.at[f]]

        def fire_nf(ci):
            blk = jnp.minimum(wid * NCHUNK + ci, NBLK - 1)
            pltpu.async_copy(nf_hbm.at[blk], nfv.at[ci % 2], isem)

        def acc_slice(bank):
            return acc.at[pl.ds(bank * NS * C + sid * C, C), :]

        def out_slices(ci):
            blk = wid * NCHUNK + ci
            return acc_slice(ci % 2), out_hbm.at[pl.ds(blk * C, C), :]

        fire_nf(0)

        def chunk_body(ci, carry):
            bank = ci % 2
            pltpu.make_async_copy(nf_hbm.at[0], nfv.at[bank], isem).wait()
            @pl.when(ci >= 2)
            def _():
                src, dst = out_slices(ci - 2)
                pltpu.make_async_copy(src, dst, osem).wait()

            for f in range(NT):
                for t in range(C // 16):
                    sl = pl.ds(t * 16, 16)
                    if f < NP:
                        a = nfv[bank, 2 * f, sl]
                        b = nfv[bank, 2 * f + 1, sl]
                        idxv[f, sl] = a * VOCAB + b + f * (VOCAB * VOCAB)
                    else:
                        idxv[f, sl] = nfv[bank, 2 * NP, sl]
                pltpu.async_copy(table_ref(f), gbuf.at[f], gsems[f])
            fire_nf(ci + 1)
            pltpu.make_async_copy(table_ref(0), gbuf.at[0], gsems[0]).wait()
            pltpu.async_copy(gbuf.at[0], acc.at[idconst.at[bank]], ssems[0])
            pltpu.make_async_copy(gbuf.at[0], acc.at[idconst.at[bank]],
                                  ssems[0]).wait()
            for f in range(1, NT):
                pltpu.make_async_copy(table_ref(f), gbuf.at[f], gsems[f]).wait()
                pltpu.async_copy(gbuf.at[f], acc.at[idconst.at[bank]], ssems[f],
                                 add=True)
            for f in range(1, NT):
                pltpu.make_async_copy(gbuf.at[f], acc.at[idconst.at[bank]],
                                      ssems[f]).wait()
            src, dst = out_slices(ci)
            pltpu.async_copy(src, dst, osem)
            return carry

        lax.fori_loop(0, NCHUNK, chunk_body, 0)

        for ci in (NCHUNK - 2, NCHUNK - 1):
            src, dst = out_slices(ci)
            pltpu.make_async_copy(src, dst, osem).wait()
        pltpu.make_async_copy(nf_hbm.at[0], nfv.at[NCHUNK % 2], isem).wait()

    return emb_kernel


@jax.jit
def kernel(node_feature, W):
    nf = node_feature.astype(jnp.int32).reshape(NBLK, C, F).transpose(0, 2, 1)
    pairs = _build_pair_tables()(
        W[0:2 * NP:2].reshape(NP, VOCAB, 1, EMB),
        W[1:2 * NP:2].reshape(NP, 1, VOCAB, EMB),
    ).reshape(NP * VOCAB * VOCAB, EMB)
    out = _build_sc_kernel()(nf, pairs, W[2 * NP])
    return out.reshape(B, N, EMB)


# speedup vs baseline: 1.4080x; 1.0982x over previous
import functools

import jax
import jax.numpy as jnp
from jax import lax
from jax.experimental import pallas as pl
from jax.experimental.pallas import tpu as pltpu
from jax.experimental.pallas import tpu_sc as plsc

B, N, F = 4096, 64, 9
VOCAB, EMB = 100, 128
R = B * N
NC, NS = 2, 16
NW = NC * NS
C = 128
NP = 4
NT = NP + 1
RSC = 131072
RW = RSC // NW
NCHUNK = RW // C
NBLK = RSC // C
TCR = 512


def _pair_body(w0_ref, w1_ref, out_ref):
    out_ref[...] = w0_ref[...] + w1_ref[...]


@functools.cache
def _build_pair_tables():
    return pl.pallas_call(
        _pair_body,
        grid=(NP,),
        in_specs=[
            pl.BlockSpec((1, VOCAB, 1, EMB), lambda p: (p, 0, 0, 0)),
            pl.BlockSpec((1, 1, VOCAB, EMB), lambda p: (p, 0, 0, 0)),
        ],
        out_specs=pl.BlockSpec((1, VOCAB, VOCAB, EMB), lambda p: (p, 0, 0, 0)),
        out_shape=jax.ShapeDtypeStruct((NP, VOCAB, VOCAB, EMB), jnp.float32),
    )


def _onehot_body(nf_ref, w_ref, out_ref):
    iota = lax.broadcasted_iota(jnp.int32, (1, 128), 1)
    oh = [
        (nf_ref[:, f][:, None] == iota).astype(jnp.float32) for f in range(F)
    ]
    ohc = jnp.concatenate(oh, axis=1)
    out_ref[...] = jnp.dot(ohc, w_ref[...], preferred_element_type=jnp.float32)


@functools.cache
def _build_onehot_kernel():
    rows_tc = R - RSC
    return pl.pallas_call(
        _onehot_body,
        grid=(rows_tc // TCR,),
        in_specs=[
            pl.BlockSpec((TCR, F), lambda i: (i, 0)),
            pl.BlockSpec((F * 128, EMB), lambda i: (0, 0)),
        ],
        out_specs=pl.BlockSpec((TCR, EMB), lambda i: (i, 0)),
        out_shape=jax.ShapeDtypeStruct((rows_tc, EMB), jnp.float32),
    )


@functools.cache
def _build_sc_kernel():
    mesh = plsc.VectorSubcoreMesh(core_axis_name="c", subcore_axis_name="s")

    @functools.partial(
        pl.kernel,
        out_type=jax.ShapeDtypeStruct((RSC, EMB), jnp.float32),
        mesh=mesh,
        scratch_types=[
            pltpu.VMEM((2, F, C), jnp.int32),
            pltpu.VMEM((NT, C), jnp.int32),
            pltpu.VMEM((NT, C, EMB), jnp.float32),
            pltpu.VMEM((2, C), jnp.int32),
            pltpu.VMEM_SHARED((2 * NS * C, EMB), jnp.float32),
            pltpu.SemaphoreType.DMA,
            pltpu.SemaphoreType.DMA,
            pltpu.SemaphoreType.DMA,
            pltpu.SemaphoreType.DMA,
            pltpu.SemaphoreType.DMA,
            pltpu.SemaphoreType.DMA,
            pltpu.SemaphoreType.DMA,
            pltpu.SemaphoreType.DMA,
            pltpu.SemaphoreType.DMA,
            pltpu.SemaphoreType.DMA,
            pltpu.SemaphoreType.DMA,
            pltpu.SemaphoreType.DMA,
        ],
    )
    def emb_kernel(nf_hbm, pair_hbm, w8_hbm, out_hbm,
                   nfv, idxv, gbuf, idconst, acc,
                   isem, osem, g0, g1, g2, g3, g4, s0, s1, s2, s3, s4):
        gsems = (g0, g1, g2, g3, g4)
        ssems = (s0, s1, s2, s3, s4)
        cid = lax.axis_index("c")
        sid = lax.axis_index("s")
        wid = sid * NC + cid
        lane = lax.broadcasted_iota(jnp.int32, (16,), 0)
        for b in range(2):
            for i in range(C // 16):
                idconst[b, pl.ds(i * 16, 16)] = lane + (b * NS * C + sid * C + i * 16)

        def table_ref(f):
            src = pair_hbm if f < NP else w8_hbm
            return src.at[idxv.at[f]]

        def fire_nf(ci):
            blk = jnp.minimum(wid * NCHUNK + ci, NBLK - 1)
            pltpu.async_copy(nf_hbm.at[blk], nfv.at[ci % 2], isem)

        def acc_slice(bank):
            return acc.at[pl.ds(bank * NS * C + sid * C, C), :]

        def out_slices(ci):
            blk = wid * NCHUNK + ci
            return acc_slice(ci % 2), out_hbm.at[pl.ds(blk * C, C), :]

        fire_nf(0)

        def chunk_body(ci, carry):
            bank = ci % 2
            pltpu.make_async_copy(nf_hbm.at[0], nfv.at[bank], isem).wait()
            @pl.when(ci >= 2)
            def _():
                src, dst = out_slices(ci - 2)
                pltpu.make_async_copy(src, dst, osem).wait()

            for f in range(NT):
                for t in range(C // 16):
                    sl = pl.ds(t * 16, 16)
                    if f < NP:
                        a = nfv[bank, 2 * f, sl]
                        b = nfv[bank, 2 * f + 1, sl]
                        idxv[f, sl] = a * VOCAB + b + f * (VOCAB * VOCAB)
                    else:
                        idxv[f, sl] = nfv[bank, 2 * NP, sl]
                pltpu.async_copy(table_ref(f), gbuf.at[f], gsems[f])
            fire_nf(ci + 1)
            pltpu.make_async_copy(table_ref(0), gbuf.at[0], gsems[0]).wait()
            pltpu.async_copy(gbuf.at[0], acc.at[idconst.at[bank]], ssems[0])
            pltpu.make_async_copy(gbuf.at[0], acc.at[idconst.at[bank]],
                                  ssems[0]).wait()
            for f in range(1, NT):
                pltpu.make_async_copy(table_ref(f), gbuf.at[f], gsems[f]).wait()
                pltpu.async_copy(gbuf.at[f], acc.at[idconst.at[bank]], ssems[f],
                                 add=True)
            for f in range(1, NT):
                pltpu.make_async_copy(gbuf.at[f], acc.at[idconst.at[bank]],
                                      ssems[f]).wait()
            src, dst = out_slices(ci)
            pltpu.async_copy(src, dst, osem)
            return carry

        lax.fori_loop(0, NCHUNK, chunk_body, 0)

        for ci in (NCHUNK - 2, NCHUNK - 1):
            src, dst = out_slices(ci)
            pltpu.make_async_copy(src, dst, osem).wait()
        pltpu.make_async_copy(nf_hbm.at[0], nfv.at[NCHUNK % 2], isem).wait()

    return emb_kernel


@jax.jit
def kernel(node_feature, W):
    nf = node_feature.astype(jnp.int32).reshape(R, F)
    nf_sc = nf[:RSC].reshape(NBLK, C, F).transpose(0, 2, 1)
    pairs = _build_pair_tables()(
        W[0:2 * NP:2].reshape(NP, VOCAB, 1, EMB),
        W[1:2 * NP:2].reshape(NP, 1, VOCAB, EMB),
    ).reshape(NP * VOCAB * VOCAB, EMB)
    sc_out = _build_sc_kernel()(nf_sc, pairs, W[2 * NP])
    wpad = jnp.pad(W, ((0, 0), (0, 128 - VOCAB), (0, 0))).reshape(F * 128, EMB)
    tc_out = _build_onehot_kernel()(nf[RSC:], wpad)
    return jnp.concatenate([sc_out, tc_out], axis=0).reshape(B, N, EMB)


# speedup vs baseline: 1.6248x; 1.1540x over previous
import functools

import jax
import jax.numpy as jnp
from jax import lax
from jax.experimental import pallas as pl
from jax.experimental.pallas import tpu as pltpu
from jax.experimental.pallas import tpu_sc as plsc

B, N, F = 4096, 64, 9
VOCAB, EMB = 100, 128
R = B * N
NC, NS = 2, 16
NW = NC * NS
C = 128
NP = 4
NT = NP + 1
RSC = 131072
RW = RSC // NW
NCHUNK = RW // C
NBLK = RSC // C
TCR = 1024


def _pair_body(w0_ref, w1_ref, out_ref):
    out_ref[...] = w0_ref[...] + w1_ref[...]


@functools.cache
def _build_pair_tables():
    return pl.pallas_call(
        _pair_body,
        grid=(NP,),
        in_specs=[
            pl.BlockSpec((1, VOCAB, 1, EMB), lambda p: (p, 0, 0, 0)),
            pl.BlockSpec((1, 1, VOCAB, EMB), lambda p: (p, 0, 0, 0)),
        ],
        out_specs=pl.BlockSpec((1, VOCAB, VOCAB, EMB), lambda p: (p, 0, 0, 0)),
        out_shape=jax.ShapeDtypeStruct((NP, VOCAB, VOCAB, EMB), jnp.float32),
    )


def _onehot_body(nf_ref, w_ref, out_ref):
    iota = lax.broadcasted_iota(jnp.int32, (1, 128), 1)
    oh = [
        (nf_ref[:, f][:, None] == iota).astype(jnp.bfloat16) for f in range(F)
    ]
    ohc = jnp.concatenate(oh, axis=1)
    out_ref[...] = jnp.dot(ohc, w_ref[...], preferred_element_type=jnp.float32)


@functools.cache
def _build_onehot_kernel():
    rows_tc = R - RSC
    return pl.pallas_call(
        _onehot_body,
        grid=(rows_tc // TCR,),
        in_specs=[
            pl.BlockSpec((TCR, F), lambda i: (i, 0)),
            pl.BlockSpec((F * 128, EMB), lambda i: (0, 0)),
        ],
        out_specs=pl.BlockSpec((TCR, EMB), lambda i: (i, 0)),
        out_shape=jax.ShapeDtypeStruct((rows_tc, EMB), jnp.float32),
    )


@functools.cache
def _build_sc_kernel():
    mesh = plsc.VectorSubcoreMesh(core_axis_name="c", subcore_axis_name="s")

    @functools.partial(
        pl.kernel,
        out_type=jax.ShapeDtypeStruct((RSC, EMB), jnp.float32),
        mesh=mesh,
        scratch_types=[
            pltpu.VMEM((2, F, C), jnp.int32),
            pltpu.VMEM((NT, C), jnp.int32),
            pltpu.VMEM((NT, C, EMB), jnp.float32),
            pltpu.VMEM((2, C), jnp.int32),
            pltpu.VMEM_SHARED((2 * NS * C, EMB), jnp.float32),
            pltpu.SemaphoreType.DMA,
            pltpu.SemaphoreType.DMA,
            pltpu.SemaphoreType.DMA,
            pltpu.SemaphoreType.DMA,
            pltpu.SemaphoreType.DMA,
            pltpu.SemaphoreType.DMA,
            pltpu.SemaphoreType.DMA,
            pltpu.SemaphoreType.DMA,
            pltpu.SemaphoreType.DMA,
            pltpu.SemaphoreType.DMA,
            pltpu.SemaphoreType.DMA,
            pltpu.SemaphoreType.DMA,
        ],
    )
    def emb_kernel(nf_hbm, pair_hbm, w8_hbm, out_hbm,
                   nfv, idxv, gbuf, idconst, acc,
                   isem, osem, g0, g1, g2, g3, g4, s0, s1, s2, s3, s4):
        gsems = (g0, g1, g2, g3, g4)
        ssems = (s0, s1, s2, s3, s4)
        cid = lax.axis_index("c")
        sid = lax.axis_index("s")
        wid = sid * NC + cid
        lane = lax.broadcasted_iota(jnp.int32, (16,), 0)
        for b in range(2):
            for i in range(C // 16):
                idconst[b, pl.ds(i * 16, 16)] = lane + (b * NS * C + sid * C + i * 16)

        def table_ref(f):
            src = pair_hbm if f < NP else w8_hbm
            return src.at[idxv.at[f]]

        def fire_nf(ci):
            blk = jnp.minimum(wid * NCHUNK + ci, NBLK - 1)
            pltpu.async_copy(nf_hbm.at[blk], nfv.at[ci % 2], isem)

        def acc_slice(bank):
            return acc.at[pl.ds(bank * NS * C + sid * C, C), :]

        def out_slices(ci):
            blk = wid * NCHUNK + ci
            return acc_slice(ci % 2), out_hbm.at[pl.ds(blk * C, C), :]

        fire_nf(0)

        def chunk_body(ci, carry):
            bank = ci % 2
            pltpu.make_async_copy(nf_hbm.at[0], nfv.at[bank], isem).wait()
            @pl.when(ci >= 2)
            def _():
                src, dst = out_slices(ci - 2)
                pltpu.make_async_copy(src, dst, osem).wait()

            for f in range(NT):
                for t in range(C // 16):
                    sl = pl.ds(t * 16, 16)
                    if f < NP:
                        a = nfv[bank, 2 * f, sl]
                        b = nfv[bank, 2 * f + 1, sl]
                        idxv[f, sl] = a * VOCAB + b + f * (VOCAB * VOCAB)
                    else:
                        idxv[f, sl] = nfv[bank, 2 * NP, sl]
                pltpu.async_copy(table_ref(f), gbuf.at[f], gsems[f])
            fire_nf(ci + 1)
            pltpu.make_async_copy(table_ref(0), gbuf.at[0], gsems[0]).wait()
            pltpu.async_copy(gbuf.at[0], acc.at[idconst.at[bank]], ssems[0])
            pltpu.make_async_copy(gbuf.at[0], acc.at[idconst.at[bank]],
                                  ssems[0]).wait()
            for f in range(1, NT):
                pltpu.make_async_copy(table_ref(f), gbuf.at[f], gsems[f]).wait()
                pltpu.async_copy(gbuf.at[f], acc.at[idconst.at[bank]], ssems[f],
                                 add=True)
            for f in range(1, NT):
                pltpu.make_async_copy(gbuf.at[f], acc.at[idconst.at[bank]],
                                      ssems[f]).wait()
            src, dst = out_slices(ci)
            pltpu.async_copy(src, dst, osem)
            return carry

        lax.fori_loop(0, NCHUNK, chunk_body, 0)

        for ci in (NCHUNK - 2, NCHUNK - 1):
            src, dst = out_slices(ci)
            pltpu.make_async_copy(src, dst, osem).wait()
        pltpu.make_async_copy(nf_hbm.at[0], nfv.at[NCHUNK % 2], isem).wait()

    return emb_kernel


@jax.jit
def kernel(node_feature, W):
    nf = node_feature.astype(jnp.int32).reshape(R, F)
    nf_sc = nf[:RSC].reshape(NBLK, C, F).transpose(0, 2, 1)
    pairs = _build_pair_tables()(
        W[0:2 * NP:2].reshape(NP, VOCAB, 1, EMB),
        W[1:2 * NP:2].reshape(NP, 1, VOCAB, EMB),
    ).reshape(NP * VOCAB * VOCAB, EMB)
    sc_out = _build_sc_kernel()(nf_sc, pairs, W[2 * NP])
    wpad = jnp.pad(W, ((0, 0), (0, 128 - VOCAB), (0, 0))) \
        .reshape(F * 128, EMB).astype(jnp.bfloat16)
    tc_out = _build_onehot_kernel()(nf[RSC:], wpad)
    return jnp.concatenate([sc_out, tc_out], axis=0).reshape(B, N, EMB)
